# Initial kernel scaffold; baseline (speedup 1.0000x reference)
import jax
import jax.numpy as jnp
from jax.experimental import pallas as pl


def kernel(z, codebook):
    raise NotImplementedError("write your pallas kernel here")


# speedup vs baseline: 1.4968x; 1.4968x over previous
import jax
import jax.numpy as jnp
from jax.experimental import pallas as pl
from jax.experimental.pallas import tpu as pltpu

_D = 64
_K = 1024
_BB = 8


def _vq_kernel(x_ref, cbt_ref, csq_ref, out_ref):
    x = x_ref[...]
    cbt = cbt_ref[...]
    scores = jnp.dot(x, cbt, preferred_element_type=jnp.float32,
                     precision=jax.lax.Precision.DEFAULT)
    x_sq = jnp.sum(x * x, axis=1, keepdims=True)
    dist = (x_sq + csq_ref[...]) - 2.0 * scores
    m = jnp.min(dist, axis=1, keepdims=True)
    iota = jax.lax.broadcasted_iota(jnp.int32, dist.shape, 1)
    idx = jnp.min(jnp.where(dist == m, iota, _K), axis=1)
    out_ref[...] = idx.reshape(out_ref.shape).astype(jnp.int32)


def kernel(z, codebook):
    B, L = z.shape
    P = L // _D
    x = z.reshape(B * P, _D)
    cbt = codebook.T
    csq = jnp.sum(codebook * codebook, axis=1)[None, :]

    bm = _BB * P
    grid = (B // _BB,)
    tokens = pl.pallas_call(
        _vq_kernel,
        grid=grid,
        in_specs=[
            pl.BlockSpec((bm, _D), lambda i: (i, 0)),
            pl.BlockSpec((_D, _K), lambda i: (0, 0)),
            pl.BlockSpec((1, _K), lambda i: (0, 0)),
        ],
        out_specs=pl.BlockSpec((_BB, P), lambda i: (i, 0)),
        out_shape=jax.ShapeDtypeStruct((B, P), jnp.int32),
    )(x, cbt, csq)
    return tokens


# speedup vs baseline: 1.8364x; 1.2269x over previous
import jax
import jax.numpy as jnp
from jax.experimental import pallas as pl
from jax.experimental.pallas import tpu as pltpu

_D = 64
_K = 1024
_BB = 8


def _vq_kernel(x_ref, cbt_ref, hcsq_ref, iota_ref, out_ref):
    x = x_ref[...]
    cbt = cbt_ref[...]
    scores = jnp.dot(x, cbt, preferred_element_type=jnp.float32,
                     precision=jax.lax.Precision.DEFAULT)
    t = scores - hcsq_ref[...]
    m = jnp.max(t, axis=1, keepdims=True)
    idx = jnp.min(jnp.where(t == m, iota_ref[...], jnp.float32(_K)), axis=1)
    out_ref[...] = idx.reshape(out_ref.shape).astype(jnp.int32)


def kernel(z, codebook):
    B, L = z.shape
    P = L // _D
    x = z.reshape(B * P, _D)
    cbt = codebook.T
    hcsq = 0.5 * jnp.sum(codebook * codebook, axis=1)[None, :]
    iota = jnp.arange(_K, dtype=jnp.float32)[None, :]

    bm = _BB * P
    grid = (B // _BB,)
    tokens = pl.pallas_call(
        _vq_kernel,
        grid=grid,
        in_specs=[
            pl.BlockSpec((bm, _D), lambda i: (i, 0)),
            pl.BlockSpec((_D, _K), lambda i: (0, 0)),
            pl.BlockSpec((1, _K), lambda i: (0, 0)),
            pl.BlockSpec((1, _K), lambda i: (0, 0)),
        ],
        out_specs=pl.BlockSpec((_BB, P), lambda i: (i, 0)),
        out_shape=jax.ShapeDtypeStruct((B, P), jnp.int32),
    )(x, cbt, hcsq, iota)
    return tokens
